# Initial kernel scaffold; baseline (speedup 1.0000x reference)
import functools

import jax
import jax.numpy as jnp
from jax import lax
from jax.experimental import pallas as pl
from jax.experimental.pallas import tpu as pltpu
from jax.experimental.pallas import tpu_sc as plsc

N = 10000
NP = 10240
E = 320000
DIN = 128
F1 = 512
F2 = 256
F3 = 64
G = 64
NC = 10

NCORES = 2
NSUB = 16
NW = NCORES * NSUB
K = 80
ROWS_PT = NP // NSUB


def _sc_mesh():
    return plsc.VectorSubcoreMesh(core_axis_name="c", subcore_axis_name="s")


@functools.partial(
    pl.kernel,
    out_type=jax.ShapeDtypeStruct((NW, NP), jnp.float32),
    mesh=_sc_mesh(),
    scratch_types=[
        pltpu.VMEM((NP,), jnp.float32),
        pltpu.VMEM((K,), jnp.int32),
    ],
)
def _deg_kernel(dst_hbm, out_hbm, deg_v, idx_v):
    cid = lax.axis_index("c")
    sid = lax.axis_index("s")
    wid = cid * NSUB + sid
    ept = E // NW

    def zero_body(i, carry):
        deg_v[pl.ds(i * 16, 16)] = jnp.zeros((16,), jnp.float32)
        return carry

    lax.fori_loop(0, NP // 16, zero_body, 0)

    ones = jnp.ones((16,), jnp.float32)

    def chunk_body(j, carry):
        base = wid * ept + j * K
        pltpu.sync_copy(dst_hbm.at[pl.ds(base, K)], idx_v)

        def sub(kk, c2):
            idx = idx_v[pl.ds(kk * 16, 16)]
            plsc.addupdate_scatter(deg_v, [idx], ones)
            return c2

        lax.fori_loop(0, K // 16, sub, 0)
        return carry

    lax.fori_loop(0, ept // K, chunk_body, 0)
    pltpu.sync_copy(deg_v, out_hbm.at[wid])


def _make_prop(F, feature_split):
    ept = E // (NSUB if feature_split else NW)
    nchunks = ept // K

    @functools.partial(
        pl.kernel,
        out_type=[
            jax.ShapeDtypeStruct((NP, F), jnp.float32),
            jax.ShapeDtypeStruct((NP, F), jnp.float32),
        ],
        mesh=_sc_mesh(),
        scratch_types=[
            pltpu.VMEM((K,), jnp.int32),
            pltpu.VMEM((K,), jnp.int32),
            pltpu.VMEM((K, F), jnp.float32),
            pltpu.VMEM_SHARED((NP, F), jnp.float32),
            pltpu.SemaphoreType.DMA,
        ],
    )
    def prop(u0_hbm, u1_hbm, src_hbm, dst_hbm, out0_hbm, out1_hbm,
             src_v, dst_v, rows_v, acc_sh, sem):
        cid = lax.axis_index("c")
        sid = lax.axis_index("s")

        def zero_body(i, carry):
            for jj in range(F // 16):
                rows_v[i, pl.ds(jj * 16, 16)] = jnp.zeros((16,), jnp.float32)
            return carry

        lax.fori_loop(0, K, zero_body, 0)
        r0 = sid * ROWS_PT
        for t in range(ROWS_PT // K):
            pltpu.sync_copy(rows_v, acc_sh.at[pl.ds(r0 + t * K, K)])
        plsc.subcore_barrier()

        if feature_split:
            tile_base = sid * ept
        else:
            tile_base = (cid * NSUB + sid) * ept

        def run_edges(u_hbm):
            def body(j, carry):
                base = tile_base + j * K
                pltpu.sync_copy(src_hbm.at[pl.ds(base, K)], src_v)
                pltpu.sync_copy(dst_hbm.at[pl.ds(base, K)], dst_v)
                pltpu.async_copy(u_hbm.at[src_v], rows_v, sem).wait()
                pltpu.sync_copy(rows_v, acc_sh.at[dst_v], add=True)
                return carry

            lax.fori_loop(0, nchunks, body, 0)

        @pl.when(cid == 0)
        def _():
            run_edges(u0_hbm)

        @pl.when(cid == 1)
        def _():
            run_edges(u1_hbm)

        plsc.subcore_barrier()

        @pl.when(cid == 0)
        def _():
            pltpu.sync_copy(acc_sh.at[pl.ds(r0, ROWS_PT)],
                            out0_hbm.at[pl.ds(r0, ROWS_PT)])

        @pl.when(cid == 1)
        def _():
            pltpu.sync_copy(acc_sh.at[pl.ds(r0, ROWS_PT)],
                            out1_hbm.at[pl.ds(r0, ROWS_PT)])

    return prop


_prop1 = _make_prop(DIN, feature_split=False)
_prop2 = _make_prop(F2 // 2, feature_split=True)
_prop3 = _make_prop(F3, feature_split=False)


def _tc_scale_body(degt_ref, x_ref, dinv_ref, u1_ref):
    deg = jnp.sum(degt_ref[...], axis=1, keepdims=True) + 1.0
    dinv = lax.rsqrt(deg)
    dinv_ref[...] = dinv
    u1_ref[...] = x_ref[...] * dinv


def _tc_scale(degt, x_pad):
    return pl.pallas_call(
        _tc_scale_body,
        out_shape=[
            jax.ShapeDtypeStruct((NP, 1), jnp.float32),
            jax.ShapeDtypeStruct((NP, DIN), jnp.float32),
        ],
    )(degt, x_pad)


R1 = 2048


def _tc_mm1_body(s1a, s1b, u1, dinv, W1, b1, W2, o_a, o_b):
    a1 = (s1a[...] + s1b[...] + u1[...]) * dinv[...]
    h1 = jnp.maximum(
        jnp.dot(a1, W1[...], preferred_element_type=jnp.float32) + b1[...], 0.0)
    z2 = jnp.dot(h1, W2[...], preferred_element_type=jnp.float32)
    u2 = z2 * dinv[...]
    o_a[...] = u2[:, :F2 // 2]
    o_b[...] = u2[:, F2 // 2:]


def _tc_mm1(s1a, s1b, u1, dinv, W1, b1, W2):
    nb = NP // R1
    return pl.pallas_call(
        _tc_mm1_body,
        grid=(nb,),
        in_specs=[
            pl.BlockSpec((R1, DIN), lambda i: (i, 0)),
            pl.BlockSpec((R1, DIN), lambda i: (i, 0)),
            pl.BlockSpec((R1, DIN), lambda i: (i, 0)),
            pl.BlockSpec((R1, 1), lambda i: (i, 0)),
            pl.BlockSpec((DIN, F1), lambda i: (0, 0)),
            pl.BlockSpec((1, F1), lambda i: (0, 0)),
            pl.BlockSpec((F1, F2), lambda i: (0, 0)),
        ],
        out_specs=[
            pl.BlockSpec((R1, F2 // 2), lambda i: (i, 0)),
            pl.BlockSpec((R1, F2 // 2), lambda i: (i, 0)),
        ],
        out_shape=[
            jax.ShapeDtypeStruct((NP, F2 // 2), jnp.float32),
            jax.ShapeDtypeStruct((NP, F2 // 2), jnp.float32),
        ],
    )(s1a, s1b, u1, dinv, W1, b1, W2)


def _tc_mm3_body(s2a, s2b, u2a, u2b, dinv, b2, W3, u3_ref):
    t = jnp.concatenate([s2a[...] + u2a[...], s2b[...] + u2b[...]], axis=1)
    h2 = jnp.maximum(t * dinv[...] + b2[...], 0.0)
    z3 = jnp.dot(h2, W3[...], preferred_element_type=jnp.float32)
    u3_ref[...] = z3 * dinv[...]


def _tc_mm3(s2a, s2b, u2a, u2b, dinv, b2, W3):
    nb = NP // R1
    return pl.pallas_call(
        _tc_mm3_body,
        grid=(nb,),
        in_specs=[
            pl.BlockSpec((R1, F2 // 2), lambda i: (i, 0)),
            pl.BlockSpec((R1, F2 // 2), lambda i: (i, 0)),
            pl.BlockSpec((R1, F2 // 2), lambda i: (i, 0)),
            pl.BlockSpec((R1, F2 // 2), lambda i: (i, 0)),
            pl.BlockSpec((R1, 1), lambda i: (i, 0)),
            pl.BlockSpec((1, F2), lambda i: (0, 0)),
            pl.BlockSpec((F2, F3), lambda i: (0, 0)),
        ],
        out_specs=pl.BlockSpec((R1, F3), lambda i: (i, 0)),
        out_shape=jax.ShapeDtypeStruct((NP, F3), jnp.float32),
    )(s2a, s2b, u2a, u2b, dinv, b2, W3)


def _tc_head_body(s3a, s3b, u3, dinv, b3, batch_row, Wl, bl, out_ref):
    a3 = (s3a[...] + s3b[...] + u3[...]) * dinv[...] + b3[...]
    gids = lax.broadcasted_iota(jnp.int32, (G, NP), 0)
    oneh = (batch_row[...] == gids).astype(jnp.float32)
    sums = jnp.dot(oneh, a3, preferred_element_type=jnp.float32)
    cnt = jnp.dot(oneh, jnp.ones((NP, 1), jnp.float32),
                  preferred_element_type=jnp.float32)
    pooled = sums / jnp.maximum(cnt, 1.0)
    out_ref[...] = (
        jnp.dot(pooled, Wl[...], preferred_element_type=jnp.float32) + bl[...])


def _tc_head(s3a, s3b, u3, dinv, b3, batch_row, Wl, bl):
    return pl.pallas_call(
        _tc_head_body,
        out_shape=jax.ShapeDtypeStruct((G, NC), jnp.float32),
    )(s3a, s3b, u3, dinv, b3, batch_row, Wl, bl)


def kernel(x, edge_index, batch, W1, b1, W2, b2, W3, b3, Wl, bl):
    src = edge_index[0]
    dst = edge_index[1]
    x_pad = jnp.pad(x, ((0, NP - N), (0, 0)))
    batch_row = jnp.pad(batch, (0, NP - N), constant_values=G)[None, :]

    degl = _deg_kernel(dst)
    dinv, u1 = _tc_scale(degl.T, x_pad)

    s1a, s1b = _prop1(u1, u1, src, dst)
    u2a, u2b = _tc_mm1(s1a, s1b, u1, dinv, W1, b1[None, :], W2)

    s2a, s2b = _prop2(u2a, u2b, src, dst)
    u3 = _tc_mm3(s2a, s2b, u2a, u2b, dinv, b2[None, :], W3)

    s3a, s3b = _prop3(u3, u3, src, dst)
    return _tc_head(s3a, s3b, u3, dinv, b3[None, :], batch_row, Wl, bl)


# speedup vs baseline: 11.8747x; 11.8747x over previous
import functools

import jax
import jax.numpy as jnp
from jax import lax
from jax.experimental import pallas as pl
from jax.experimental.pallas import tpu as pltpu
from jax.experimental.pallas import tpu_sc as plsc

N = 10000
NP = 10240
E = 320000
DIN = 128
F1 = 512
F2 = 256
F3 = 64
G = 64
NC = 10

NCORES = 2
NSUB = 16
NW = NCORES * NSUB
K = 80
ROWS_PT = NP // NSUB


def _sc_mesh():
    return plsc.VectorSubcoreMesh(core_axis_name="c", subcore_axis_name="s")


DW = 128


@functools.partial(
    pl.kernel,
    out_type=[
        jax.ShapeDtypeStruct((NP, DW), jnp.float32),
        jax.ShapeDtypeStruct((NP, DW), jnp.float32),
    ],
    mesh=_sc_mesh(),
    scratch_types=[
        pltpu.VMEM((K,), jnp.int32),
        pltpu.VMEM((K, DW), jnp.float32),
        pltpu.VMEM_SHARED((NP, DW), jnp.float32),
    ],
)
def _deg_kernel(dst_hbm, out0_hbm, out1_hbm, dst_v, ones_v, acc_sh):
    cid = lax.axis_index("c")
    sid = lax.axis_index("s")
    ept = E // NW

    def zero_body(i, carry):
        ones_v[i, pl.ds(0, DW)] = jnp.zeros((DW,), jnp.float32)
        return carry

    lax.fori_loop(0, K, zero_body, 0)
    r0 = sid * ROWS_PT
    for t in range(ROWS_PT // K):
        pltpu.sync_copy(ones_v, acc_sh.at[pl.ds(r0 + t * K, K)])

    def fill_body(i, carry):
        ones_v[i, pl.ds(0, DW)] = jnp.ones((DW,), jnp.float32)
        return carry

    lax.fori_loop(0, K, fill_body, 0)
    plsc.subcore_barrier()

    tile_base = (cid * NSUB + sid) * ept

    def chunk_body(j, carry):
        base = tile_base + j * K
        pltpu.sync_copy(dst_hbm.at[pl.ds(base, K)], dst_v)
        pltpu.sync_copy(ones_v, acc_sh.at[dst_v], add=True)
        return carry

    lax.fori_loop(0, ept // K, chunk_body, 0)
    plsc.subcore_barrier()

    @pl.when(cid == 0)
    def _():
        pltpu.sync_copy(acc_sh.at[pl.ds(r0, ROWS_PT)],
                        out0_hbm.at[pl.ds(r0, ROWS_PT)])

    @pl.when(cid == 1)
    def _():
        pltpu.sync_copy(acc_sh.at[pl.ds(r0, ROWS_PT)],
                        out1_hbm.at[pl.ds(r0, ROWS_PT)])


def _make_prop(F, feature_split):
    ept = E // (NSUB if feature_split else NW)
    nchunks = ept // K

    @functools.partial(
        pl.kernel,
        out_type=[
            jax.ShapeDtypeStruct((NP, F), jnp.float32),
            jax.ShapeDtypeStruct((NP, F), jnp.float32),
        ],
        mesh=_sc_mesh(),
        scratch_types=[
            pltpu.VMEM((K,), jnp.int32),
            pltpu.VMEM((K,), jnp.int32),
            pltpu.VMEM((K, F), jnp.float32),
            pltpu.VMEM_SHARED((NP, F), jnp.float32),
            pltpu.SemaphoreType.DMA,
        ],
    )
    def prop(u0_hbm, u1_hbm, src_hbm, dst_hbm, out0_hbm, out1_hbm,
             src_v, dst_v, rows_v, acc_sh, sem):
        cid = lax.axis_index("c")
        sid = lax.axis_index("s")

        def zero_body(i, carry):
            for jj in range(F // 16):
                rows_v[i, pl.ds(jj * 16, 16)] = jnp.zeros((16,), jnp.float32)
            return carry

        lax.fori_loop(0, K, zero_body, 0)
        r0 = sid * ROWS_PT
        for t in range(ROWS_PT // K):
            pltpu.sync_copy(rows_v, acc_sh.at[pl.ds(r0 + t * K, K)])
        plsc.subcore_barrier()

        if feature_split:
            tile_base = sid * ept
        else:
            tile_base = (cid * NSUB + sid) * ept

        def run_edges(u_hbm):
            def body(j, carry):
                base = tile_base + j * K
                pltpu.sync_copy(src_hbm.at[pl.ds(base, K)], src_v)
                pltpu.sync_copy(dst_hbm.at[pl.ds(base, K)], dst_v)
                pltpu.async_copy(u_hbm.at[src_v], rows_v, sem).wait()
                pltpu.sync_copy(rows_v, acc_sh.at[dst_v], add=True)
                return carry

            lax.fori_loop(0, nchunks, body, 0)

        @pl.when(cid == 0)
        def _():
            run_edges(u0_hbm)

        @pl.when(cid == 1)
        def _():
            run_edges(u1_hbm)

        plsc.subcore_barrier()

        @pl.when(cid == 0)
        def _():
            pltpu.sync_copy(acc_sh.at[pl.ds(r0, ROWS_PT)],
                            out0_hbm.at[pl.ds(r0, ROWS_PT)])

        @pl.when(cid == 1)
        def _():
            pltpu.sync_copy(acc_sh.at[pl.ds(r0, ROWS_PT)],
                            out1_hbm.at[pl.ds(r0, ROWS_PT)])

    return prop


_prop1 = _make_prop(DIN, feature_split=False)
_prop2 = _make_prop(F2 // 2, feature_split=True)
_prop3 = _make_prop(DIN, feature_split=False)


def _tc_scale_body(d0_ref, d1_ref, x_ref, dinv_ref, u1_ref):
    deg = d0_ref[:, :1] + d1_ref[:, :1] + 1.0
    dinv = lax.rsqrt(deg)
    dinv_ref[...] = dinv
    u1_ref[...] = x_ref[...] * dinv


def _tc_scale(d0, d1, x_pad):
    return pl.pallas_call(
        _tc_scale_body,
        out_shape=[
            jax.ShapeDtypeStruct((NP, 1), jnp.float32),
            jax.ShapeDtypeStruct((NP, DIN), jnp.float32),
        ],
    )(d0, d1, x_pad)


R1 = 2048


def _tc_mm1_body(s1a, s1b, u1, dinv, W1, b1, W2, o_a, o_b):
    a1 = (s1a[...] + s1b[...] + u1[...]) * dinv[...]
    h1 = jnp.maximum(
        jnp.dot(a1, W1[...], preferred_element_type=jnp.float32) + b1[...], 0.0)
    z2 = jnp.dot(h1, W2[...], preferred_element_type=jnp.float32)
    u2 = z2 * dinv[...]
    o_a[...] = u2[:, :F2 // 2]
    o_b[...] = u2[:, F2 // 2:]


def _tc_mm1(s1a, s1b, u1, dinv, W1, b1, W2):
    nb = NP // R1
    return pl.pallas_call(
        _tc_mm1_body,
        grid=(nb,),
        in_specs=[
            pl.BlockSpec((R1, DIN), lambda i: (i, 0)),
            pl.BlockSpec((R1, DIN), lambda i: (i, 0)),
            pl.BlockSpec((R1, DIN), lambda i: (i, 0)),
            pl.BlockSpec((R1, 1), lambda i: (i, 0)),
            pl.BlockSpec((DIN, F1), lambda i: (0, 0)),
            pl.BlockSpec((1, F1), lambda i: (0, 0)),
            pl.BlockSpec((F1, F2), lambda i: (0, 0)),
        ],
        out_specs=[
            pl.BlockSpec((R1, F2 // 2), lambda i: (i, 0)),
            pl.BlockSpec((R1, F2 // 2), lambda i: (i, 0)),
        ],
        out_shape=[
            jax.ShapeDtypeStruct((NP, F2 // 2), jnp.float32),
            jax.ShapeDtypeStruct((NP, F2 // 2), jnp.float32),
        ],
    )(s1a, s1b, u1, dinv, W1, b1, W2)


def _tc_mm3_body(s2a, s2b, u2a, u2b, dinv, b2, W3, u3_ref):
    t = jnp.concatenate([s2a[...] + u2a[...], s2b[...] + u2b[...]], axis=1)
    h2 = jnp.maximum(t * dinv[...] + b2[...], 0.0)
    z3 = jnp.dot(h2, W3[...], preferred_element_type=jnp.float32)
    u3 = z3 * dinv[...]
    u3_ref[...] = jnp.concatenate(
        [u3, jnp.zeros((u3.shape[0], DIN - F3), jnp.float32)], axis=1)


def _tc_mm3(s2a, s2b, u2a, u2b, dinv, b2, W3):
    nb = NP // R1
    return pl.pallas_call(
        _tc_mm3_body,
        grid=(nb,),
        in_specs=[
            pl.BlockSpec((R1, F2 // 2), lambda i: (i, 0)),
            pl.BlockSpec((R1, F2 // 2), lambda i: (i, 0)),
            pl.BlockSpec((R1, F2 // 2), lambda i: (i, 0)),
            pl.BlockSpec((R1, F2 // 2), lambda i: (i, 0)),
            pl.BlockSpec((R1, 1), lambda i: (i, 0)),
            pl.BlockSpec((1, F2), lambda i: (0, 0)),
            pl.BlockSpec((F2, F3), lambda i: (0, 0)),
        ],
        out_specs=pl.BlockSpec((R1, DIN), lambda i: (i, 0)),
        out_shape=jax.ShapeDtypeStruct((NP, DIN), jnp.float32),
    )(s2a, s2b, u2a, u2b, dinv, b2, W3)


def _tc_head_body(s3a, s3b, u3, dinv, b3, batch_row, Wl, bl, out_ref):
    a3 = ((s3a[...] + s3b[...] + u3[...]) * dinv[...])[:, :F3] + b3[...]
    gids = lax.broadcasted_iota(jnp.int32, (G, NP), 0)
    oneh = (batch_row[...] == gids).astype(jnp.float32)
    sums = jnp.dot(oneh, a3, preferred_element_type=jnp.float32)
    cnt = jnp.dot(oneh, jnp.ones((NP, 1), jnp.float32),
                  preferred_element_type=jnp.float32)
    pooled = sums / jnp.maximum(cnt, 1.0)
    out_ref[...] = (
        jnp.dot(pooled, Wl[...], preferred_element_type=jnp.float32) + bl[...])


def _tc_head(s3a, s3b, u3, dinv, b3, batch_row, Wl, bl):
    return pl.pallas_call(
        _tc_head_body,
        out_shape=jax.ShapeDtypeStruct((G, NC), jnp.float32),
    )(s3a, s3b, u3, dinv, b3, batch_row, Wl, bl)


def kernel(x, edge_index, batch, W1, b1, W2, b2, W3, b3, Wl, bl):
    src = edge_index[0]
    dst = edge_index[1]
    x_pad = jnp.pad(x, ((0, NP - N), (0, 0)))
    batch_row = jnp.pad(batch, (0, NP - N), constant_values=G)[None, :]

    d0, d1 = _deg_kernel(dst)
    dinv, u1 = _tc_scale(d0, d1, x_pad)

    s1a, s1b = _prop1(u1, u1, src, dst)
    u2a, u2b = _tc_mm1(s1a, s1b, u1, dinv, W1, b1[None, :], W2)

    s2a, s2b = _prop2(u2a, u2b, src, dst)
    u3 = _tc_mm3(s2a, s2b, u2a, u2b, dinv, b2[None, :], W3)

    s3a, s3b = _prop3(u3, u3, src, dst)
    return _tc_head(s3a, s3b, u3, dinv, b3[None, :], batch_row, Wl, bl)
